# Initial kernel scaffold; baseline (speedup 1.0000x reference)
import jax
import jax.numpy as jnp
from jax.experimental import pallas as pl


def kernel(pre, y_label, stage_name):
    raise NotImplementedError("write your pallas kernel here")


# speedup vs baseline: 1.2547x; 1.2547x over previous
import jax
import jax.numpy as jnp
from jax import lax
from jax.experimental import pallas as pl


_ROWS_PER_BLK = 1024
_NC = 3


def _body(pre_ref, lab_ref, score_ref, pred_oh_ref, lab_oh_ref):
    x = pre_ref[...]
    R = x.shape[0]
    lane = lax.broadcasted_iota(jnp.int32, x.shape, 1)
    valid = lane < _NC
    neg_inf = jnp.float32(-jnp.inf)
    xm = jnp.where(valid, x, neg_inf)
    m = jnp.max(xm, axis=1, keepdims=True)
    e = jnp.where(valid, jnp.exp(x - m), 0.0)
    s = jnp.sum(e, axis=1, keepdims=True)
    y = e / s

    big = jnp.int32(10**6)
    idx = jnp.where(valid & (xm == m), lane, big)
    pred = jnp.min(idx, axis=1, keepdims=True)

    lane3 = lax.broadcasted_iota(jnp.int32, (R, _NC), 1)
    score_ref[...] = y[:, :_NC]
    pred_oh_ref[...] = (lane3 == pred).astype(jnp.float32)
    lab = lab_ref[...]
    lab_oh_ref[...] = (lane3 == lab).astype(jnp.float32)


def kernel(pre, y_label, stage_name):
    n, _ = pre.shape
    grid = n // _ROWS_PER_BLK
    lab2d = y_label.reshape(n, 1).astype(jnp.int32)
    out_shapes = (
        jax.ShapeDtypeStruct((n, _NC), jnp.float32),
        jax.ShapeDtypeStruct((n, _NC), jnp.float32),
        jax.ShapeDtypeStruct((n, _NC), jnp.float32),
    )
    o_spec = pl.BlockSpec((_ROWS_PER_BLK, _NC), lambda i: (i, 0))
    return pl.pallas_call(
        _body,
        grid=(grid,),
        in_specs=[
            pl.BlockSpec((_ROWS_PER_BLK, 128), lambda i: (i, 0)),
            pl.BlockSpec((_ROWS_PER_BLK, 1), lambda i: (i, 0)),
        ],
        out_specs=(o_spec, o_spec, o_spec),
        out_shape=out_shapes,
    )(pre, lab2d)
